# baseline (speedup 1.0000x reference)
import jax
import jax.numpy as jnp
from jax import lax
from jax.experimental import pallas as pl
from jax.experimental.pallas import tpu as pltpu
from jax.experimental.pallas import tpu_sc as plsc

BATCH = 4096
SEQ = 200
EMBED = 64
NC, NS, LANES = 2, 16, 16
NW = NC * NS
BROWS = BATCH // NW
IDX_PER_W = BROWS * SEQ
NBUF = 4
TCHUNK = 32


def _body(idx_hbm, tok_hbm, pos_hbm, out_hbm,
          idx_c, idx_t, pos_v, gbufs, stages, gsems, osems):
    wid = lax.axis_index("s") * NC + lax.axis_index("c")
    base = wid * IDX_PER_W
    pltpu.sync_copy(pos_hbm, pos_v)

    iota = lax.iota(jnp.int32, LANES)
    iota_seq = iota * SEQ

    for q in range(BROWS // TCHUNK):
        pltpu.sync_copy(
            idx_hbm.at[pl.ds(base + q * TCHUNK * SEQ, TCHUNK * SEQ)], idx_c
        )

        @pl.loop(0, SEQ)
        def _tr(s):
            for g in range(TCHUNK // LANES):
                vals = plsc.load_gather(idx_c, [iota_seq + (g * LANES * SEQ + s)])
                idx_t[s, pl.ds((q * TCHUNK) + g * LANES, LANES)] = vals

    def gather(s, b):
        pltpu.async_copy(tok_hbm.at[idx_t.at[s]], gbufs.at[b], gsems[b])

    def gather_wait(s, b):
        pltpu.make_async_copy(
            tok_hbm.at[idx_t.at[s]], gbufs.at[b], gsems[b]
        ).wait()

    def put(s, b):
        pltpu.async_copy(stages.at[b], out_hbm.at[s, :, wid], osems[b])

    def put_wait(s, b):
        pltpu.make_async_copy(
            stages.at[b], out_hbm.at[s, :, wid], osems[b]
        ).wait()

    for b in range(NBUF - 1):
        gather(b, b)

    @pl.loop(0, SEQ, step=NBUF)
    def _outer(k0):
        for bi in range(NBUF):
            s = k0 + bi
            gather_wait(s, bi)

            bn = (bi + NBUF - 1) % NBUF

            @pl.when(s + NBUF - 1 < SEQ)
            def _prefetch():
                @pl.when(s >= 1)
                def _drain():
                    put_wait(s - 1, bn)

                gather(s + NBUF - 1, bn)

            s_splat = jnp.full((LANES,), 0, jnp.int32) + s
            bi_splat = jnp.full((LANES,), bi, jnp.int32)

            @pl.loop(0, LANES)
            def _diag(d):
                rot = (iota + d) & (LANES - 1)
                for e0 in range(0, EMBED, LANES):
                    e_idx = e0 + rot
                    pv = plsc.load_gather(pos_v, [s_splat, e_idx])
                    i_hi = e_idx // 8
                    i_lo = e_idx % 8
                    for r0 in range(0, BROWS, LANES):
                        b_idx = iota + r0
                        tv = plsc.load_gather(gbufs, [bi_splat, b_idx, e_idx])
                        plsc.store_scatter(
                            stages.at[bi], [i_hi, i_lo, b_idx], tv + pv
                        )

            put(s, bi)

    for b in range(NBUF):
        put_wait(SEQ - NBUF + b, (SEQ - NBUF + b) % NBUF)


def kernel(inputs, token_table, pos_table):
    flat_idx = inputs.reshape(-1).astype(jnp.int32)
    mesh = plsc.VectorSubcoreMesh(core_axis_name="c", subcore_axis_name="s")
    out = pl.kernel(
        _body,
        out_type=jax.ShapeDtypeStruct((SEQ, EMBED // 8, NW, 8, 128), jnp.float32),
        mesh=mesh,
        scratch_types=[
            pltpu.VMEM((TCHUNK * SEQ,), jnp.int32),
            pltpu.VMEM((SEQ, BROWS), jnp.int32),
            pltpu.VMEM((SEQ, EMBED), jnp.float32),
            pltpu.VMEM((NBUF, BROWS, EMBED), jnp.float32),
            pltpu.VMEM((NBUF, EMBED // 8, 8, 128), jnp.float32),
            [pltpu.SemaphoreType.DMA] * NBUF,
            [pltpu.SemaphoreType.DMA] * NBUF,
        ],
        compiler_params=pltpu.CompilerParams(
            use_tc_tiling_on_sc=False, needs_layout_passes=False
        ),
    )(flat_idx, token_table, pos_table)
    return out.transpose(2, 4, 0, 1, 3).reshape(BATCH, SEQ, EMBED)


# speedup vs baseline: 1.1964x; 1.1964x over previous
import jax
import jax.numpy as jnp
from jax import lax
from jax.experimental import pallas as pl
from jax.experimental.pallas import tpu as pltpu
from jax.experimental.pallas import tpu_sc as plsc

BATCH = 4096
SEQ = 200
EMBED = 64
NC, NS, LANES = 2, 16, 16
NW = NC * NS
BROWS = BATCH // NW
IDX_PER_W = BROWS * SEQ
NBUF = 4
TCHUNK = 32


def _body(idx_hbm, tok_hbm, pos_hbm, out_hbm,
          idx_c, idx_t, pos_v, gbufs, stages, gsems, osems):
    wid = lax.axis_index("s") * NC + lax.axis_index("c")
    base = wid * IDX_PER_W
    pltpu.sync_copy(pos_hbm, pos_v)

    iota = lax.iota(jnp.int32, LANES)
    iota_seq = iota * SEQ

    for q in range(BROWS // TCHUNK):
        pltpu.sync_copy(
            idx_hbm.at[pl.ds(base + q * TCHUNK * SEQ, TCHUNK * SEQ)], idx_c
        )

        @pl.loop(0, SEQ)
        def _tr(s):
            for g in range(TCHUNK // LANES):
                vals = plsc.load_gather(idx_c, [iota_seq + (g * LANES * SEQ + s)])
                idx_t[s, pl.ds((q * TCHUNK) + g * LANES, LANES)] = vals

    def gather(s, b):
        pltpu.async_copy(tok_hbm.at[idx_t.at[s]], gbufs.at[b], gsems[b])

    def gather_wait(s, b):
        pltpu.make_async_copy(
            tok_hbm.at[idx_t.at[s]], gbufs.at[b], gsems[b]
        ).wait()

    def put(s, b):
        pltpu.async_copy(stages.at[b], out_hbm.at[s, :, wid], osems[b])

    def put_wait(s, b):
        pltpu.make_async_copy(
            stages.at[b], out_hbm.at[s, :, wid], osems[b]
        ).wait()

    for b in range(NBUF - 1):
        gather(b, b)

    @pl.loop(0, SEQ, step=NBUF)
    def _outer(k0):
        for bi in range(NBUF):
            s = k0 + bi
            gather_wait(s, bi)

            bn = (bi + NBUF - 1) % NBUF

            @pl.when(s + NBUF - 1 < SEQ)
            def _prefetch():
                gather(s + NBUF - 1, bn)

            @pl.when(s >= NBUF)
            def _drain():
                put_wait(s - NBUF, bi)

            s_splat = jnp.full((LANES,), 0, jnp.int32) + s
            bi_splat = jnp.full((LANES,), bi, jnp.int32)

            @pl.loop(0, LANES, unroll=2)
            def _diag(d):
                rot = (iota + d) & (LANES - 1)
                for e0 in range(0, EMBED, LANES):
                    e_idx = e0 + rot
                    pv = plsc.load_gather(pos_v, [s_splat, e_idx])
                    i_hi = e_idx // 8
                    i_lo = e_idx % 8
                    for r0 in range(0, BROWS, LANES):
                        b_idx = iota + r0
                        tv = plsc.load_gather(gbufs, [bi_splat, b_idx, e_idx])
                        plsc.store_scatter(
                            stages.at[bi], [i_hi, i_lo, b_idx], tv + pv
                        )

            put(s, bi)

    for b in range(NBUF):
        put_wait(SEQ - NBUF + b, (SEQ - NBUF + b) % NBUF)


def kernel(inputs, token_table, pos_table):
    flat_idx = inputs.reshape(-1).astype(jnp.int32)
    mesh = plsc.VectorSubcoreMesh(core_axis_name="c", subcore_axis_name="s")
    out = pl.kernel(
        _body,
        out_type=jax.ShapeDtypeStruct((SEQ, EMBED // 8, NW, 8, 128), jnp.float32),
        mesh=mesh,
        scratch_types=[
            pltpu.VMEM((TCHUNK * SEQ,), jnp.int32),
            pltpu.VMEM((SEQ, BROWS), jnp.int32),
            pltpu.VMEM((SEQ, EMBED), jnp.float32),
            pltpu.VMEM((NBUF, BROWS, EMBED), jnp.float32),
            pltpu.VMEM((NBUF, EMBED // 8, 8, 128), jnp.float32),
            [pltpu.SemaphoreType.DMA] * NBUF,
            [pltpu.SemaphoreType.DMA] * NBUF,
        ],
        compiler_params=pltpu.CompilerParams(
            use_tc_tiling_on_sc=False, needs_layout_passes=False
        ),
    )(flat_idx, token_table, pos_table)
    return out.transpose(2, 4, 0, 1, 3).reshape(BATCH, SEQ, EMBED)
